# baseline (speedup 1.0000x reference)
import functools
import math

import jax
import jax.numpy as jnp
import numpy as np
from jax import lax
from jax.experimental import pallas as pl
from jax.experimental.pallas import tpu as pltpu
from jax.experimental.pallas import tpu_sc as plsc

_SPARSITY = 0.05
_L = 16
_UNROLL = 16

_SIGN = np.int32(-(2**31))
_ONE = np.int32(1)
_ZERO = np.int32(0)


def _lane_sum(v):
    s = v[0]
    for lane in range(1, _L):
        s = s + v[lane]
    return s


def _lane_max(v):
    m = v[0]
    for lane in range(1, _L):
        m = jnp.maximum(m, v[lane])
    return m


def _bits2key(b):
    return jnp.where(b >= 0, b, jnp.bitwise_not(b) ^ _SIGN)


def _key2f_scalar(s):
    b = jnp.where(s >= 0, s, jnp.bitwise_not(s ^ _SIGN))
    return lax.bitcast_convert_type(b, jnp.float32)


def _f2key_scalar(f):
    b = lax.bitcast_convert_type(f, jnp.int32)
    return jnp.where(b >= 0, b, jnp.bitwise_not(b) ^ _SIGN)


@functools.partial(jax.jit, static_argnums=(1, 2, 3))
def _kwta_sc(xi, B, N, k):
    n_chunks = N // (_L * _UNROLL)

    mesh = plsc.VectorSubcoreMesh(core_axis_name="c", subcore_axis_name="s")

    @functools.partial(
        pl.kernel,
        mesh=mesh,
        out_type=jax.ShapeDtypeStruct((B, N), jnp.float32),
        scratch_types=[
            pltpu.VMEM((N,), jnp.int32),
            pltpu.VMEM((N,), jnp.float32),
        ],
    )
    def kwta(xi_hbm, out_hbm, kbuf, obuf):
        wid = lax.axis_index("s") * 2 + lax.axis_index("c")
        rows_per_w = B // 32

        def count_ge(c):
            def body(j, acc):
                base = j * (_L * _UNROLL)
                for u in range(_UNROLL):
                    v = kbuf[pl.ds(base + u * _L, _L)]
                    acc = acc + jnp.where(v >= c, _ONE, _ZERO)
                return acc
            acc = lax.fori_loop(0, n_chunks, body,
                                jnp.zeros((_L,), jnp.int32))
            return _lane_sum(acc)

        for r in range(rows_per_w):
            row = wid * rows_per_w + r
            pltpu.sync_copy(xi_hbm.at[row], kbuf)

            def conv_body(j, acc):
                base = j * (_L * _UNROLL)
                for u in range(_UNROLL):
                    b = kbuf[pl.ds(base + u * _L, _L)]
                    s = _bits2key(b)
                    kbuf[pl.ds(base + u * _L, _L)] = s
                    acc = acc + jnp.where(s >= 0, _ONE, _ZERO)
                return acc
            acc0 = lax.fori_loop(0, n_chunks, conv_body,
                                 jnp.zeros((_L,), jnp.int32))
            cnt0 = _lane_sum(acc0)
            p = jnp.where(cnt0 >= k, np.int32(0), _SIGN)

            def bit_body(i, p):
                c = p | (np.int32(1) << (np.int32(30) - i))
                cnt = count_ge(c)
                return jnp.where(cnt >= k, c, p)
            p = lax.fori_loop(0, 31, bit_body, p)

            def low_body(j, carry):
                acc, mx = carry
                base = j * (_L * _UNROLL)
                for u in range(_UNROLL):
                    v = kbuf[pl.ds(base + u * _L, _L)]
                    ge = v >= p
                    acc = acc + jnp.where(ge, _ONE, _ZERO)
                    mx = jnp.maximum(mx, jnp.where(ge, _SIGN, v))
                return acc, mx
            accg, mxv = lax.fori_loop(
                0, n_chunks, low_body,
                (jnp.zeros((_L,), jnp.int32),
                 jnp.full((_L,), _SIGN, jnp.int32)))
            cnt_ge = _lane_sum(accg)
            max_low = _lane_max(mxv)
            s2 = jnp.where(cnt_ge >= k + 1, p, max_low)

            thr = (_key2f_scalar(p) + _key2f_scalar(s2)) * np.float32(0.5)
            tkey = jnp.where(thr == np.float32(0.0), np.int32(0),
                             _f2key_scalar(thr))

            def mask_body(j, _):
                base = j * (_L * _UNROLL)
                for u in range(_UNROLL):
                    v = kbuf[pl.ds(base + u * _L, _L)]
                    obuf[pl.ds(base + u * _L, _L)] = jnp.where(
                        v > tkey, np.float32(1.0), np.float32(0.0))
                return 0
            lax.fori_loop(0, n_chunks, mask_body, 0)

            pltpu.sync_copy(obuf, out_hbm.at[row])

    return kwta(xi)


def kernel(x):
    B, N = x.shape
    k = math.ceil(_SPARSITY * N)
    if k == N:
        k -= 1
    xi = lax.bitcast_convert_type(x, jnp.int32)
    return _kwta_sc(xi, B, N, k)


# speedup vs baseline: 1.0234x; 1.0234x over previous
import functools
import math

import jax
import jax.numpy as jnp
import numpy as np
from jax import lax
from jax.experimental import pallas as pl
from jax.experimental.pallas import tpu as pltpu
from jax.experimental.pallas import tpu_sc as plsc

_SPARSITY = 0.05
_L = 16
_UNROLL = 8
_NACC = 4

_SIGN = np.int32(-(2**31))
_ONE = np.int32(1)
_ZERO = np.int32(0)


def _lane_sum(v):
    s = v[0]
    for lane in range(1, _L):
        s = s + v[lane]
    return s


def _lane_max(v):
    m = v[0]
    for lane in range(1, _L):
        m = jnp.maximum(m, v[lane])
    return m


def _bits2key(b):
    return jnp.where(b >= 0, b, jnp.bitwise_not(b) ^ _SIGN)


def _key2f_scalar(s):
    b = jnp.where(s >= 0, s, jnp.bitwise_not(s ^ _SIGN))
    return lax.bitcast_convert_type(b, jnp.float32)


def _f2key_scalar(f):
    b = lax.bitcast_convert_type(f, jnp.int32)
    return jnp.where(b >= 0, b, jnp.bitwise_not(b) ^ _SIGN)


@functools.partial(jax.jit, static_argnums=(1, 2, 3))
def _kwta_sc(xi, B, N, k):
    n_chunks = N // (_L * _UNROLL)

    mesh = plsc.VectorSubcoreMesh(core_axis_name="c", subcore_axis_name="s")

    @functools.partial(
        pl.kernel,
        mesh=mesh,
        out_type=jax.ShapeDtypeStruct((B, N), jnp.float32),
        scratch_types=[
            pltpu.VMEM((N,), jnp.int32),
            pltpu.VMEM((N,), jnp.float32),
        ],
    )
    def kwta(xi_hbm, out_hbm, kbuf, obuf):
        wid = lax.axis_index("s") * 2 + lax.axis_index("c")
        rows_per_w = B // 32

        def count_ge(c):
            def body(j, accs):
                base = j * (_L * _UNROLL)
                accs = list(accs)
                for u in range(_UNROLL):
                    v = kbuf[pl.ds(base + u * _L, _L)]
                    a = u % _NACC
                    accs[a] = accs[a] + jnp.where(v >= c, _ONE, _ZERO)
                return tuple(accs)
            accs = lax.fori_loop(
                0, n_chunks, body,
                tuple(jnp.zeros((_L,), jnp.int32) for _ in range(_NACC)))
            tot = accs[0]
            for a in range(1, _NACC):
                tot = tot + accs[a]
            return _lane_sum(tot)

        for r in range(rows_per_w):
            row = wid * rows_per_w + r
            pltpu.sync_copy(xi_hbm.at[row], kbuf)

            def conv_body(j, accs):
                base = j * (_L * _UNROLL)
                accs = list(accs)
                for u in range(_UNROLL):
                    b = kbuf[pl.ds(base + u * _L, _L)]
                    s = _bits2key(b)
                    kbuf[pl.ds(base + u * _L, _L)] = s
                    a = u % _NACC
                    accs[a] = accs[a] + jnp.where(s >= 0, _ONE, _ZERO)
                return tuple(accs)
            accs0 = lax.fori_loop(
                0, n_chunks, conv_body,
                tuple(jnp.zeros((_L,), jnp.int32) for _ in range(_NACC)))
            tot0 = accs0[0]
            for a in range(1, _NACC):
                tot0 = tot0 + accs0[a]
            cnt0 = _lane_sum(tot0)
            p = jnp.where(cnt0 >= k, np.int32(0), _SIGN)

            def bit_body(i, p):
                c = p | (np.int32(1) << (np.int32(30) - i))
                cnt = count_ge(c)
                return jnp.where(cnt >= k, c, p)
            p = lax.fori_loop(0, 31, bit_body, p)

            def low_body(j, carry):
                accs, mxs = list(carry[0]), list(carry[1])
                base = j * (_L * _UNROLL)
                for u in range(_UNROLL):
                    v = kbuf[pl.ds(base + u * _L, _L)]
                    ge = v >= p
                    a = u % _NACC
                    accs[a] = accs[a] + jnp.where(ge, _ONE, _ZERO)
                    mxs[a] = jnp.maximum(mxs[a], jnp.where(ge, _SIGN, v))
                return tuple(accs), tuple(mxs)
            accsg, mxvs = lax.fori_loop(
                0, n_chunks, low_body,
                (tuple(jnp.zeros((_L,), jnp.int32) for _ in range(_NACC)),
                 tuple(jnp.full((_L,), _SIGN, jnp.int32)
                       for _ in range(_NACC))))
            totg, totm = accsg[0], mxvs[0]
            for a in range(1, _NACC):
                totg = totg + accsg[a]
                totm = jnp.maximum(totm, mxvs[a])
            cnt_ge = _lane_sum(totg)
            max_low = _lane_max(totm)
            s2 = jnp.where(cnt_ge >= k + 1, p, max_low)

            thr = (_key2f_scalar(p) + _key2f_scalar(s2)) * np.float32(0.5)
            tkey = jnp.where(thr == np.float32(0.0), np.int32(0),
                             _f2key_scalar(thr))

            def mask_body(j, _):
                base = j * (_L * _UNROLL)
                for u in range(_UNROLL):
                    v = kbuf[pl.ds(base + u * _L, _L)]
                    obuf[pl.ds(base + u * _L, _L)] = jnp.where(
                        v > tkey, np.float32(1.0), np.float32(0.0))
                return 0
            lax.fori_loop(0, n_chunks, mask_body, 0)

            pltpu.sync_copy(obuf, out_hbm.at[row])

    return kwta(xi)


def kernel(x):
    B, N = x.shape
    k = math.ceil(_SPARSITY * N)
    if k == N:
        k -= 1
    xi = lax.bitcast_convert_type(x, jnp.int32)
    return _kwta_sc(xi, B, N, k)


# speedup vs baseline: 1.3451x; 1.3143x over previous
import functools
import math

import jax
import jax.numpy as jnp
import numpy as np
from jax import lax
from jax.experimental import pallas as pl
from jax.experimental.pallas import tpu as pltpu
from jax.experimental.pallas import tpu_sc as plsc

_SPARSITY = 0.05
_L = 16
_UNROLL = 8
_NACC = 4

_SIGN = np.int32(-(2**31))
_ONE = np.int32(1)
_ZERO = np.int32(0)
_NEGINF = np.float32(-np.inf)


def _lane_sum(v):
    s = v[0]
    for lane in range(1, _L):
        s = s + v[lane]
    return s


def _lane_max(v):
    m = v[0]
    for lane in range(1, _L):
        m = jnp.maximum(m, v[lane])
    return m


def _key2f_scalar(s):
    b = jnp.where(s >= 0, s, jnp.bitwise_not(s ^ _SIGN))
    return lax.bitcast_convert_type(b, jnp.float32)


def _kwta_sc_rows(x, B_sc, N, k):
    n_chunks = N // (_L * _UNROLL)

    mesh = plsc.VectorSubcoreMesh(core_axis_name="c", subcore_axis_name="s")

    @functools.partial(
        pl.kernel,
        mesh=mesh,
        out_type=jax.ShapeDtypeStruct((B_sc, N), jnp.float32),
        scratch_types=[
            pltpu.VMEM((N,), jnp.float32),
            pltpu.VMEM((N,), jnp.float32),
        ],
    )
    def kwta(x_hbm, out_hbm, xbuf, obuf):
        wid = lax.axis_index("s") * 2 + lax.axis_index("c")
        rows_per_w = B_sc // 32

        def count_ge(fc):
            def body(j, accs):
                base = j * (_L * _UNROLL)
                accs = list(accs)
                for u in range(_UNROLL):
                    v = xbuf[pl.ds(base + u * _L, _L)]
                    a = u % _NACC
                    accs[a] = accs[a] + jnp.where(v >= fc, _ONE, _ZERO)
                return tuple(accs)
            accs = lax.fori_loop(
                0, n_chunks, body,
                tuple(jnp.zeros((_L,), jnp.int32) for _ in range(_NACC)))
            tot = accs[0]
            for a in range(1, _NACC):
                tot = tot + accs[a]
            return _lane_sum(tot)

        for r in range(rows_per_w):
            row = wid * rows_per_w + r
            pltpu.sync_copy(x_hbm.at[row], xbuf)

            cnt0 = count_ge(np.float32(0.0))
            p = jnp.where(cnt0 >= k, np.int32(0), _SIGN)

            def bit_body(i, p):
                c = p | (np.int32(1) << (np.int32(30) - i))
                cnt = count_ge(_key2f_scalar(c))
                return jnp.where(cnt >= k, c, p)
            p = lax.fori_loop(0, 31, bit_body, p)
            fp = _key2f_scalar(p)

            def low_body(j, carry):
                accs, mxs = list(carry[0]), list(carry[1])
                base = j * (_L * _UNROLL)
                for u in range(_UNROLL):
                    v = xbuf[pl.ds(base + u * _L, _L)]
                    ge = v >= fp
                    a = u % _NACC
                    accs[a] = accs[a] + jnp.where(ge, _ONE, _ZERO)
                    mxs[a] = jnp.maximum(mxs[a], jnp.where(ge, _NEGINF, v))
                return tuple(accs), tuple(mxs)
            accsg, mxvs = lax.fori_loop(
                0, n_chunks, low_body,
                (tuple(jnp.zeros((_L,), jnp.int32) for _ in range(_NACC)),
                 tuple(jnp.full((_L,), _NEGINF, jnp.float32)
                       for _ in range(_NACC))))
            totg, totm = accsg[0], mxvs[0]
            for a in range(1, _NACC):
                totg = totg + accsg[a]
                totm = jnp.maximum(totm, mxvs[a])
            cnt_ge = _lane_sum(totg)
            max_low = _lane_max(totm)
            s2 = jnp.where(cnt_ge >= k + 1, fp, max_low)

            thr = (fp + s2) * np.float32(0.5)

            def mask_body(j, _):
                base = j * (_L * _UNROLL)
                for u in range(_UNROLL):
                    v = xbuf[pl.ds(base + u * _L, _L)]
                    obuf[pl.ds(base + u * _L, _L)] = jnp.where(
                        v > thr, np.float32(1.0), np.float32(0.0))
                return 0
            lax.fori_loop(0, n_chunks, mask_body, 0)

            pltpu.sync_copy(obuf, out_hbm.at[row])

    return kwta(x)


def _kwta_tc_rows(x, k):
    B_tc, N = x.shape

    def body(x_ref, o_ref, kref):
        xb = x_ref[...]
        bits = pltpu.bitcast(xb, jnp.int32)
        keys = jnp.where(bits >= 0, bits, jnp.bitwise_not(bits) ^ _SIGN)
        kref[...] = keys

        def count_ge(c):
            m = keys >= c
            return jnp.sum(jnp.where(m, _ONE, _ZERO), axis=1, keepdims=True)

        cnt0 = count_ge(jnp.zeros((B_tc, 1), jnp.int32))
        p = jnp.where(cnt0 >= k, np.int32(0), _SIGN)

        def bit_body(i, p):
            c = p | (np.int32(1) << (np.int32(30) - i))
            cnt = count_ge(c)
            return jnp.where(cnt >= k, c, p)
        p = lax.fori_loop(0, 31, bit_body, p, unroll=True)

        ge = keys >= p
        cnt_ge = jnp.sum(jnp.where(ge, _ONE, _ZERO), axis=1, keepdims=True)
        low = jnp.where(ge, _SIGN, keys)
        max_low = jnp.max(low, axis=1, keepdims=True)
        s2 = jnp.where(cnt_ge >= k + 1, p, max_low)

        def k2f(s):
            b = jnp.where(s >= 0, s, jnp.bitwise_not(s ^ _SIGN))
            return pltpu.bitcast(b, jnp.float32)

        thr = (k2f(p) + k2f(s2)) * np.float32(0.5)
        o_ref[...] = jnp.where(xb > thr, np.float32(1.0), np.float32(0.0))

    return pl.pallas_call(
        body,
        out_shape=jax.ShapeDtypeStruct((B_tc, N), jnp.float32),
        scratch_shapes=[pltpu.VMEM((B_tc, N), jnp.int32)],
    )(x)


@functools.partial(jax.jit, static_argnums=(1, 2, 3))
def _kwta(x, B, N, k):
    b_sc = 32 if B > 32 else B
    out_sc = _kwta_sc_rows(x[:b_sc], b_sc, N, k)
    if b_sc == B:
        return out_sc
    out_tc = _kwta_tc_rows(x[b_sc:], k)
    return jnp.concatenate([out_sc, out_tc], axis=0)


def kernel(x):
    B, N = x.shape
    k = math.ceil(_SPARSITY * N)
    if k == N:
        k -= 1
    return _kwta(x, B, N, k)


# speedup vs baseline: 1.3553x; 1.0076x over previous
import functools
import math

import jax
import jax.numpy as jnp
import numpy as np
from jax import lax
from jax.experimental import pallas as pl
from jax.experimental.pallas import tpu as pltpu
from jax.experimental.pallas import tpu_sc as plsc

_SPARSITY = 0.05
_L = 16
_UNROLL = 8
_NACC = 4

_SIGN = np.int32(-(2**31))
_ONE = np.int32(1)
_ZERO = np.int32(0)
_NEGINF = np.float32(-np.inf)


def _lane_sum(v):
    s = v[0]
    for lane in range(1, _L):
        s = s + v[lane]
    return s


def _lane_max(v):
    m = v[0]
    for lane in range(1, _L):
        m = jnp.maximum(m, v[lane])
    return m


def _key2f_scalar(s):
    b = jnp.where(s >= 0, s, jnp.bitwise_not(s ^ _SIGN))
    return lax.bitcast_convert_type(b, jnp.float32)


def _kwta_sc_rows(x, B, B_sc, N, k):
    n_chunks = N // (_L * _UNROLL)

    mesh = plsc.VectorSubcoreMesh(core_axis_name="c", subcore_axis_name="s")

    @functools.partial(
        pl.kernel,
        mesh=mesh,
        out_type=jax.ShapeDtypeStruct((B_sc, N), jnp.float32),
        scratch_types=[
            pltpu.VMEM((N,), jnp.float32),
            pltpu.VMEM((N,), jnp.float32),
        ],
    )
    def kwta(x_hbm, out_hbm, xbuf, obuf):
        wid = lax.axis_index("s") * 2 + lax.axis_index("c")
        rows_per_w = B_sc // 32

        def count_ge(fc):
            def body(j, accs):
                base = j * (_L * _UNROLL)
                accs = list(accs)
                for u in range(_UNROLL):
                    v = xbuf[pl.ds(base + u * _L, _L)]
                    a = u % _NACC
                    accs[a] = accs[a] + jnp.where(v >= fc, _ONE, _ZERO)
                return tuple(accs)
            accs = lax.fori_loop(
                0, n_chunks, body,
                tuple(jnp.zeros((_L,), jnp.int32) for _ in range(_NACC)))
            tot = accs[0]
            for a in range(1, _NACC):
                tot = tot + accs[a]
            return _lane_sum(tot)

        for r in range(rows_per_w):
            row = wid * rows_per_w + r
            pltpu.sync_copy(x_hbm.at[row], xbuf)

            cnt0 = count_ge(np.float32(0.0))
            p = jnp.where(cnt0 >= k, np.int32(0), _SIGN)

            def bit_body(i, p):
                c = p | (np.int32(1) << (np.int32(30) - i))
                cnt = count_ge(_key2f_scalar(c))
                return jnp.where(cnt >= k, c, p)
            p = lax.fori_loop(0, 31, bit_body, p)
            fp = _key2f_scalar(p)

            def low_body(j, carry):
                accs, mxs = list(carry[0]), list(carry[1])
                base = j * (_L * _UNROLL)
                for u in range(_UNROLL):
                    v = xbuf[pl.ds(base + u * _L, _L)]
                    ge = v >= fp
                    a = u % _NACC
                    accs[a] = accs[a] + jnp.where(ge, _ONE, _ZERO)
                    mxs[a] = jnp.maximum(mxs[a], jnp.where(ge, _NEGINF, v))
                return tuple(accs), tuple(mxs)
            accsg, mxvs = lax.fori_loop(
                0, n_chunks, low_body,
                (tuple(jnp.zeros((_L,), jnp.int32) for _ in range(_NACC)),
                 tuple(jnp.full((_L,), _NEGINF, jnp.float32)
                       for _ in range(_NACC))))
            totg, totm = accsg[0], mxvs[0]
            for a in range(1, _NACC):
                totg = totg + accsg[a]
                totm = jnp.maximum(totm, mxvs[a])
            cnt_ge = _lane_sum(totg)
            max_low = _lane_max(totm)
            s2 = jnp.where(cnt_ge >= k + 1, fp, max_low)

            thr = (fp + s2) * np.float32(0.5)

            def mask_body(j, _):
                base = j * (_L * _UNROLL)
                for u in range(_UNROLL):
                    v = xbuf[pl.ds(base + u * _L, _L)]
                    obuf[pl.ds(base + u * _L, _L)] = jnp.where(
                        v > thr, np.float32(1.0), np.float32(0.0))
                return 0
            lax.fori_loop(0, n_chunks, mask_body, 0)

            pltpu.sync_copy(obuf, out_hbm.at[row])

    return kwta(x)


def _kwta_tc_rows(x, B_tc, k):
    B, N = x.shape

    def body(x_ref, o_ref, kref):
        xb = x_ref[...]
        bits = pltpu.bitcast(xb, jnp.int32)
        keys = jnp.where(bits >= 0, bits, jnp.bitwise_not(bits) ^ _SIGN)
        kref[...] = keys

        def count_ge(c):
            m = keys >= c
            return jnp.sum(jnp.where(m, _ONE, _ZERO), axis=1, keepdims=True)

        cnt0 = count_ge(jnp.zeros((B_tc, 1), jnp.int32))
        p = jnp.where(cnt0 >= k, np.int32(0), _SIGN)

        def bit_body(i, p):
            c = p | (np.int32(1) << (np.int32(30) - i))
            cnt = count_ge(c)
            return jnp.where(cnt >= k, c, p)
        p = lax.fori_loop(0, 31, bit_body, p, unroll=True)

        ge = keys >= p
        cnt_ge = jnp.sum(jnp.where(ge, _ONE, _ZERO), axis=1, keepdims=True)
        low = jnp.where(ge, _SIGN, keys)
        max_low = jnp.max(low, axis=1, keepdims=True)
        s2 = jnp.where(cnt_ge >= k + 1, p, max_low)

        def k2f(s):
            b = jnp.where(s >= 0, s, jnp.bitwise_not(s ^ _SIGN))
            return pltpu.bitcast(b, jnp.float32)

        thr = (k2f(p) + k2f(s2)) * np.float32(0.5)
        o_ref[...] = jnp.where(xb > thr, np.float32(1.0), np.float32(0.0))

    return pl.pallas_call(
        body,
        grid=(1,),
        in_specs=[pl.BlockSpec((B_tc, N), lambda i: (1, 0))],
        out_specs=pl.BlockSpec((B_tc, N), lambda i: (0, 0)),
        out_shape=jax.ShapeDtypeStruct((B_tc, N), jnp.float32),
        scratch_shapes=[pltpu.VMEM((B_tc, N), jnp.int32)],
    )(x)


@functools.partial(jax.jit, static_argnums=(1, 2, 3))
def _kwta(x, B, N, k):
    b_sc = 32 if B > 32 else B
    out_sc = _kwta_sc_rows(x, B, b_sc, N, k)
    if b_sc == B:
        return out_sc
    out_tc = _kwta_tc_rows(x, B - b_sc, k)
    return jnp.concatenate([out_sc, out_tc], axis=0)


def kernel(x):
    B, N = x.shape
    k = math.ceil(_SPARSITY * N)
    if k == N:
        k -= 1
    return _kwta(x, B, N, k)


# speedup vs baseline: 1.4767x; 1.0896x over previous
import functools
import math

import jax
import jax.numpy as jnp
import numpy as np
from jax import lax
from jax.experimental import pallas as pl
from jax.experimental.pallas import tpu as pltpu
from jax.experimental.pallas import tpu_sc as plsc

_SPARSITY = 0.05
_L = 16
_UNROLL = 8
_NACC = 4

_SIGN = np.int32(-(2**31))
_ONE = np.int32(1)
_ZERO = np.int32(0)
_NEGINF = np.float32(-np.inf)
_POSINF = np.float32(np.inf)


def _lane_sum(v):
    s = v[0]
    for lane in range(1, _L):
        s = s + v[lane]
    return s


def _lane_max(v):
    m = v[0]
    for lane in range(1, _L):
        m = jnp.maximum(m, v[lane])
    return m


def _lane_min(v):
    m = v[0]
    for lane in range(1, _L):
        m = jnp.minimum(m, v[lane])
    return m


def _key2f_scalar(s):
    b = jnp.where(s >= 0, s, jnp.bitwise_not(s ^ _SIGN))
    return lax.bitcast_convert_type(b, jnp.float32)


def _kwta_sc_rows(x, B, B_sc, N, k):
    n_chunks = N // (_L * _UNROLL)

    mesh = plsc.VectorSubcoreMesh(core_axis_name="c", subcore_axis_name="s")

    @functools.partial(
        pl.kernel,
        mesh=mesh,
        out_type=jax.ShapeDtypeStruct((B_sc, N), jnp.float32),
        scratch_types=[
            pltpu.VMEM((N,), jnp.float32),
            pltpu.VMEM((N,), jnp.float32),
        ],
    )
    def kwta(x_hbm, out_hbm, xbuf, obuf):
        wid = lax.axis_index("s") * 2 + lax.axis_index("c")
        rows_per_w = B_sc // 32

        def count_ge(fc, n):
            def body(j, accs):
                base = j * (_L * _UNROLL)
                accs = list(accs)
                for u in range(_UNROLL):
                    v = xbuf[pl.ds(base + u * _L, _L)]
                    a = u % _NACC
                    accs[a] = accs[a] + jnp.where(v >= fc, _ONE, _ZERO)
                return tuple(accs)
            accs = lax.fori_loop(
                0, n, body,
                tuple(jnp.zeros((_L,), jnp.int32) for _ in range(_NACC)))
            tot = accs[0]
            for a in range(1, _NACC):
                tot = tot + accs[a]
            return _lane_sum(tot)

        for r in range(rows_per_w):
            row = wid * rows_per_w + r
            pltpu.sync_copy(x_hbm.at[row], xbuf)

            cnt0 = count_ge(np.float32(0.0), n_chunks)
            p0 = jnp.where(cnt0 >= k, np.int32(0), _SIGN)

            def bit_body(i, carry):
                p, cntp = carry
                n = jnp.where(cntp == k, 0, n_chunks)
                c = p | (np.int32(1) << (np.int32(30) - i))
                cnt = count_ge(_key2f_scalar(c), n)
                take = jnp.logical_and(cntp != k, cnt >= k)
                return (jnp.where(take, c, p), jnp.where(take, cnt, cntp))
            p, cntp = lax.fori_loop(0, 31, bit_body, (p0, cnt0))
            fp = _key2f_scalar(p)

            def mm_body(j, carry):
                mns, mxs = list(carry[0]), list(carry[1])
                base = j * (_L * _UNROLL)
                for u in range(_UNROLL):
                    v = xbuf[pl.ds(base + u * _L, _L)]
                    ge = v >= fp
                    a = u % _NACC
                    mns[a] = jnp.minimum(mns[a], jnp.where(ge, v, _POSINF))
                    mxs[a] = jnp.maximum(mxs[a], jnp.where(ge, _NEGINF, v))
                return tuple(mns), tuple(mxs)
            mnvs, mxvs = lax.fori_loop(
                0, n_chunks, mm_body,
                (tuple(jnp.full((_L,), _POSINF, jnp.float32)
                       for _ in range(_NACC)),
                 tuple(jnp.full((_L,), _NEGINF, jnp.float32)
                       for _ in range(_NACC))))
            totn, totm = mnvs[0], mxvs[0]
            for a in range(1, _NACC):
                totn = jnp.minimum(totn, mnvs[a])
                totm = jnp.maximum(totm, mxvs[a])
            vk = _lane_min(totn)
            vk1 = _lane_max(totm)

            thr = jnp.where(cntp == k,
                            (vk + vk1) * np.float32(0.5), fp)

            def mask_body(j, _):
                base = j * (_L * _UNROLL)
                for u in range(_UNROLL):
                    v = xbuf[pl.ds(base + u * _L, _L)]
                    obuf[pl.ds(base + u * _L, _L)] = jnp.where(
                        v > thr, np.float32(1.0), np.float32(0.0))
                return 0
            lax.fori_loop(0, n_chunks, mask_body, 0)

            pltpu.sync_copy(obuf, out_hbm.at[row])

    return kwta(x)


def _kwta_tc_rows(x, B_tc, k):
    B, N = x.shape

    def body(x_ref, o_ref, kref):
        xb = x_ref[...]
        bits = pltpu.bitcast(xb, jnp.int32)
        keys = jnp.where(bits >= 0, bits, jnp.bitwise_not(bits) ^ _SIGN)
        kref[...] = keys

        def count_ge(c):
            m = keys >= c
            return jnp.sum(jnp.where(m, _ONE, _ZERO), axis=1, keepdims=True)

        cnt0 = count_ge(jnp.zeros((B_tc, 1), jnp.int32))
        p = jnp.where(cnt0 >= k, np.int32(0), _SIGN)

        def bit_body(i, p):
            c = p | (np.int32(1) << (np.int32(30) - i))
            cnt = count_ge(c)
            return jnp.where(cnt >= k, c, p)
        p = lax.fori_loop(0, 31, bit_body, p, unroll=True)

        ge = keys >= p
        cnt_ge = jnp.sum(jnp.where(ge, _ONE, _ZERO), axis=1, keepdims=True)
        low = jnp.where(ge, _SIGN, keys)
        max_low = jnp.max(low, axis=1, keepdims=True)
        s2 = jnp.where(cnt_ge >= k + 1, p, max_low)

        def k2f(s):
            b = jnp.where(s >= 0, s, jnp.bitwise_not(s ^ _SIGN))
            return pltpu.bitcast(b, jnp.float32)

        thr = (k2f(p) + k2f(s2)) * np.float32(0.5)
        o_ref[...] = jnp.where(xb > thr, np.float32(1.0), np.float32(0.0))

    return pl.pallas_call(
        body,
        grid=(1,),
        in_specs=[pl.BlockSpec((B_tc, N), lambda i: (1, 0))],
        out_specs=pl.BlockSpec((B_tc, N), lambda i: (0, 0)),
        out_shape=jax.ShapeDtypeStruct((B_tc, N), jnp.float32),
        scratch_shapes=[pltpu.VMEM((B_tc, N), jnp.int32)],
    )(x)


@functools.partial(jax.jit, static_argnums=(1, 2, 3))
def _kwta(x, B, N, k):
    b_sc = 32 if B > 32 else B
    out_sc = _kwta_sc_rows(x, B, b_sc, N, k)
    if b_sc == B:
        return out_sc
    out_tc = _kwta_tc_rows(x, B - b_sc, k)
    return jnp.concatenate([out_sc, out_tc], axis=0)


def kernel(x):
    B, N = x.shape
    k = math.ceil(_SPARSITY * N)
    if k == N:
        k -= 1
    return _kwta(x, B, N, k)


# speedup vs baseline: 1.5142x; 1.0254x over previous
import functools
import math

import jax
import jax.numpy as jnp
import numpy as np
from jax import lax
from jax.experimental import pallas as pl
from jax.experimental.pallas import tpu as pltpu
from jax.experimental.pallas import tpu_sc as plsc

_SPARSITY = 0.05
_L = 16
_UNROLL = 8
_NACC = 4

_SIGN = np.int32(-(2**31))
_ONE = np.int32(1)
_ZERO = np.int32(0)
_NEGINF = np.float32(-np.inf)
_POSINF = np.float32(np.inf)


def _lane_sum(v):
    s = v[0]
    for lane in range(1, _L):
        s = s + v[lane]
    return s


def _lane_max(v):
    m = v[0]
    for lane in range(1, _L):
        m = jnp.maximum(m, v[lane])
    return m


def _lane_min(v):
    m = v[0]
    for lane in range(1, _L):
        m = jnp.minimum(m, v[lane])
    return m


def _key2f_scalar(s):
    b = jnp.where(s >= 0, s, jnp.bitwise_not(s ^ _SIGN))
    return lax.bitcast_convert_type(b, jnp.float32)


def _kwta_sc_rows(x, B, B_sc, N, k):
    n_chunks = N // (_L * _UNROLL)

    mesh = plsc.VectorSubcoreMesh(core_axis_name="c", subcore_axis_name="s")

    @functools.partial(
        pl.kernel,
        mesh=mesh,
        out_type=jax.ShapeDtypeStruct((B_sc, N), jnp.float32),
        scratch_types=[
            pltpu.VMEM((N,), jnp.float32),
            pltpu.VMEM((N,), jnp.float32),
        ],
    )
    def kwta(x_hbm, out_hbm, xbuf, obuf):
        wid = lax.axis_index("s") * 2 + lax.axis_index("c")
        rows_per_w = B_sc // 32

        def count_ge(fc, n):
            def body(j, accs):
                base = j * (_L * _UNROLL)
                accs = list(accs)
                for u in range(_UNROLL):
                    v = xbuf[pl.ds(base + u * _L, _L)]
                    a = u % _NACC
                    accs[a] = accs[a] + jnp.where(v >= fc, _ONE, _ZERO)
                return tuple(accs)
            accs = lax.fori_loop(
                0, n, body,
                tuple(jnp.zeros((_L,), jnp.int32) for _ in range(_NACC)))
            tot = accs[0]
            for a in range(1, _NACC):
                tot = tot + accs[a]
            return _lane_sum(tot)

        for r in range(rows_per_w):
            row = wid * rows_per_w + r
            pltpu.sync_copy(x_hbm.at[row], xbuf)

            cnt0 = count_ge(np.float32(0.0), n_chunks)
            p0 = jnp.where(cnt0 >= k, np.int32(0), _SIGN)

            def bit_body(i, carry):
                p, cntp = carry
                n = jnp.where(cntp == k, 0, n_chunks)
                c = p | (np.int32(1) << (np.int32(30) - i))
                cnt = count_ge(_key2f_scalar(c), n)
                take = jnp.logical_and(cntp != k, cnt >= k)
                return (jnp.where(take, c, p), jnp.where(take, cnt, cntp))
            p, cntp = lax.fori_loop(0, 31, bit_body, (p0, cnt0))
            fp = _key2f_scalar(p)

            def mm_body(j, carry):
                mns, mxs = list(carry[0]), list(carry[1])
                base = j * (_L * _UNROLL)
                for u in range(_UNROLL):
                    v = xbuf[pl.ds(base + u * _L, _L)]
                    ge = v >= fp
                    a = u % _NACC
                    mns[a] = jnp.minimum(mns[a], jnp.where(ge, v, _POSINF))
                    mxs[a] = jnp.maximum(mxs[a], jnp.where(ge, _NEGINF, v))
                return tuple(mns), tuple(mxs)
            mnvs, mxvs = lax.fori_loop(
                0, n_chunks, mm_body,
                (tuple(jnp.full((_L,), _POSINF, jnp.float32)
                       for _ in range(_NACC)),
                 tuple(jnp.full((_L,), _NEGINF, jnp.float32)
                       for _ in range(_NACC))))
            totn, totm = mnvs[0], mxvs[0]
            for a in range(1, _NACC):
                totn = jnp.minimum(totn, mnvs[a])
                totm = jnp.maximum(totm, mxvs[a])
            vk = _lane_min(totn)
            vk1 = _lane_max(totm)

            thr = jnp.where(cntp == k,
                            (vk + vk1) * np.float32(0.5), fp)

            def mask_body(j, _):
                base = j * (_L * _UNROLL)
                for u in range(_UNROLL):
                    v = xbuf[pl.ds(base + u * _L, _L)]
                    obuf[pl.ds(base + u * _L, _L)] = jnp.where(
                        v > thr, np.float32(1.0), np.float32(0.0))
                return 0
            lax.fori_loop(0, n_chunks, mask_body, 0)

            pltpu.sync_copy(obuf, out_hbm.at[row])

    return kwta(x)


def _kwta_tc_rows(x, B_tc, k):
    B, N = x.shape

    def body(x_ref, o_ref, kref):
        xb = x_ref[...]
        bits = pltpu.bitcast(xb, jnp.int32)
        keys = jnp.where(bits >= 0, bits, jnp.bitwise_not(bits) ^ _SIGN)
        kref[...] = keys

        def count_ge(c):
            m = keys >= c
            return jnp.sum(jnp.where(m, _ONE, _ZERO), axis=1, keepdims=True)

        cnt0 = count_ge(jnp.zeros((B_tc, 1), jnp.int32))
        p = jnp.where(cnt0 >= k, np.int32(0), _SIGN)

        def bit_body(i, p):
            c = p | (np.int32(1) << (np.int32(30) - i))
            cnt = count_ge(c)
            return jnp.where(cnt >= k, c, p)
        p = lax.fori_loop(0, 31, bit_body, p, unroll=True)

        ge = keys >= p
        cnt_ge = jnp.sum(jnp.where(ge, _ONE, _ZERO), axis=1, keepdims=True)
        low = jnp.where(ge, _SIGN, keys)
        max_low = jnp.max(low, axis=1, keepdims=True)
        s2 = jnp.where(cnt_ge >= k + 1, p, max_low)

        def k2f(s):
            b = jnp.where(s >= 0, s, jnp.bitwise_not(s ^ _SIGN))
            return pltpu.bitcast(b, jnp.float32)

        thr = (k2f(p) + k2f(s2)) * np.float32(0.5)
        o_ref[...] = jnp.where(xb > thr, np.float32(1.0), np.float32(0.0))

    return pl.pallas_call(
        body,
        grid=(1,),
        in_specs=[pl.BlockSpec((B_tc, N), lambda i: (1, 0))],
        out_specs=pl.BlockSpec((B_tc, N), lambda i: (1, 0)),
        out_shape=jax.ShapeDtypeStruct((B, N), jnp.float32),
        scratch_shapes=[pltpu.VMEM((B_tc, N), jnp.int32)],
    )(x)


@functools.partial(jax.jit, static_argnums=(1, 2, 3))
def _kwta(x, B, N, k):
    b_sc = 32 if B > 32 else B
    out_sc = _kwta_sc_rows(x, B, b_sc, N, k)
    if b_sc == B:
        return out_sc
    out_tc_full = _kwta_tc_rows(x, B - b_sc, k)
    return lax.dynamic_update_slice(out_tc_full, out_sc, (0, 0))


def kernel(x):
    B, N = x.shape
    k = math.ceil(_SPARSITY * N)
    if k == N:
        k -= 1
    return _kwta(x, B, N, k)
